# Initial kernel scaffold; baseline (speedup 1.0000x reference)
import jax
import jax.numpy as jnp
from jax import lax
from jax.experimental import pallas as pl
from jax.experimental.pallas import tpu as pltpu
from jax.experimental.pallas import tpu_sc as plsc

B = 16384
ED = 16
EM = 16

_info = plsc.get_sparse_core_info()
_NC, _NS = _info.num_cores, _info.num_subcores
_NW = _NC * _NS
_BPW = B // _NW
_CH = 128
_NCH = _BPW // _CH


def _gather_body(dtab, mtab, didx, midx, dout, mout,
                 idx_d, idx_m, rows_d, rows_m, sem):
    wid = lax.axis_index("s") * _NC + lax.axis_index("c")
    pltpu.sync_copy(didx.at[wid], idx_d)
    pltpu.sync_copy(midx.at[wid], idx_m)
    copies = []
    for j in range(_NCH):
        copies.append(pltpu.async_copy(dtab.at[idx_d.at[j]], rows_d.at[j], sem))
        copies.append(pltpu.async_copy(mtab.at[idx_m.at[j]], rows_m.at[j], sem))
    for c in copies:
        c.wait()
    pltpu.sync_copy(rows_d, dout.at[wid])
    pltpu.sync_copy(rows_m, mout.at[wid])


_gather = pl.kernel(
    _gather_body,
    out_type=(
        jax.ShapeDtypeStruct((_NW, _NCH, _CH, ED), jnp.float32),
        jax.ShapeDtypeStruct((_NW, _NCH, _CH, EM), jnp.float32),
    ),
    mesh=plsc.VectorSubcoreMesh(core_axis_name="c", subcore_axis_name="s"),
    scratch_types=[
        pltpu.VMEM((_NCH, _CH), jnp.int32),
        pltpu.VMEM((_NCH, _CH), jnp.int32),
        pltpu.VMEM((_NCH, _CH, ED), jnp.float32),
        pltpu.VMEM((_NCH, _CH, EM), jnp.float32),
        pltpu.SemaphoreType.DMA,
    ],
)


def _bn(y, g, be, relu):
    mu = jnp.mean(y, axis=0, keepdims=True)
    var = jnp.mean((y - mu) * (y - mu), axis=0, keepdims=True)
    out = g * (y - mu) * lax.rsqrt(var + 1e-5) + be
    return jnp.maximum(out, 0.0) if relu else out


def _trunk_body(d, m, v,
                dW, db, dg, dbe, mW, mb, mg, mbe, vW, vb, vg, vbe,
                b1W1d, b1W1m, b1W1v, b1b1, b1g1, b1be1,
                b1W2, b1b2, b1g2, b1be2, b1sWd, b1sWm, b1sWv, b1sb,
                b2W1, b2b1, b2g1, b2be1, b2W2, b2b2, b2g2, b2be2, b2sW, b2sb,
                b3W1, b3b1, b3g1, b3be1, b3W2, b3b2, b3g2, b3be2, b3sW, b3sb,
                oW, ob, out):
    f32 = jnp.float32

    def mm(x, w):
        return lax.dot(x, w[...], preferred_element_type=f32)

    dh = _bn(mm(d[...], dW) + db[...], dg[...], dbe[...], True)
    mh = _bn(mm(m[...], mW) + mb[...], mg[...], mbe[...], True)
    vh = _bn(mm(v[...], vW) + vb[...], vg[...], vbe[...], True)

    y1 = mm(dh, b1W1d) + mm(mh, b1W1m) + mm(vh, b1W1v) + b1b1[...]
    h1 = _bn(y1, b1g1[...], b1be1[...], True)
    y2 = mm(h1, b1W2) + b1b2[...]
    r1 = mm(dh, b1sWd) + mm(mh, b1sWm) + mm(vh, b1sWv) + b1sb[...]
    x = jnp.maximum(_bn(y2, b1g2[...], b1be2[...], False) + r1, 0.0)

    for (W1, b1, g1, be1, W2, b2, g2, be2, sW, sb) in (
            (b2W1, b2b1, b2g1, b2be1, b2W2, b2b2, b2g2, b2be2, b2sW, b2sb),
            (b3W1, b3b1, b3g1, b3be1, b3W2, b3b2, b3g2, b3be2, b3sW, b3sb)):
        h = _bn(mm(x, W1) + b1[...], g1[...], be1[...], True)
        y = _bn(mm(h, W2) + b2[...], g2[...], be2[...], False)
        r = mm(x, sW) + sb[...]
        x = jnp.maximum(y + r, 0.0)

    out[...] = mm(x, oW) + ob[...]


def _trunk(d, m, v, *ws):
    return pl.pallas_call(
        _trunk_body,
        out_shape=jax.ShapeDtypeStruct((B, 1), jnp.float32),
    )(d, m, v, *ws)


def kernel(diag, med, vitals, params):
    p = params
    d_rows, m_rows = _gather(
        p['diag_emb'], p['med_emb'],
        diag.reshape(_NW, _NCH, _CH), med.reshape(_NW, _NCH, _CH))
    d_rows = d_rows.reshape(B, ED)
    m_rows = m_rows.reshape(B, EM)

    def head_ws(h):
        return (h['W'].T, h['b'].reshape(1, -1),
                h['g'].reshape(1, -1), h['be'].reshape(1, -1))

    def block_ws(blk):
        return (blk['W1'].T, blk['b1'].reshape(1, -1),
                blk['g1'].reshape(1, -1), blk['be1'].reshape(1, -1),
                blk['W2'].T, blk['b2'].reshape(1, -1),
                blk['g2'].reshape(1, -1), blk['be2'].reshape(1, -1),
                blk['skipW'].T, blk['skipb'].reshape(1, -1))

    b1 = p['blocks'][0]
    W1t = b1['W1'].T
    sWt = b1['skipW'].T
    ws = (*head_ws(p['diag_head']), *head_ws(p['med_head']),
          *head_ws(p['vital_head']),
          W1t[:32], W1t[32:64], W1t[64:], b1['b1'].reshape(1, -1),
          b1['g1'].reshape(1, -1), b1['be1'].reshape(1, -1),
          b1['W2'].T, b1['b2'].reshape(1, -1),
          b1['g2'].reshape(1, -1), b1['be2'].reshape(1, -1),
          sWt[:32], sWt[32:64], sWt[64:], b1['skipb'].reshape(1, -1),
          *block_ws(p['blocks'][1]), *block_ws(p['blocks'][2]),
          p['outW'].T, p['outb'].reshape(1, -1))
    return _trunk(d_rows, m_rows, vitals, *ws)


# speedup vs baseline: 1.0304x; 1.0304x over previous
import jax
import jax.numpy as jnp
from jax import lax
from jax.experimental import pallas as pl
from jax.experimental.pallas import tpu as pltpu
from jax.experimental.pallas import tpu_sc as plsc

B = 16384
ED = 16
EM = 16

_NC, _NS = 2, 16
_NW = _NC * _NS
_BPW = B // _NW
_CH = 128
_NCH = _BPW // _CH


def _gather_body(dtab, mtab, didx, midx, dout, mout,
                 idx_d, idx_m, rows_d, rows_m, sem):
    wid = lax.axis_index("s") * _NC + lax.axis_index("c")
    pltpu.sync_copy(didx.at[wid], idx_d)
    pltpu.sync_copy(midx.at[wid], idx_m)
    copies = []
    for j in range(_NCH):
        copies.append(pltpu.async_copy(dtab.at[idx_d.at[j]], rows_d.at[j], sem))
        copies.append(pltpu.async_copy(mtab.at[idx_m.at[j]], rows_m.at[j], sem))
    for c in copies:
        c.wait()
    pltpu.sync_copy(rows_d, dout.at[wid])
    pltpu.sync_copy(rows_m, mout.at[wid])


_gather_kernel_cache = []


def _gather(dtab, mtab, didx, midx):
    if not _gather_kernel_cache:
        _gather_kernel_cache.append(pl.kernel(
            _gather_body,
            out_type=(
                jax.ShapeDtypeStruct((_NW, _NCH, _CH, ED), jnp.float32),
                jax.ShapeDtypeStruct((_NW, _NCH, _CH, EM), jnp.float32),
            ),
            mesh=plsc.VectorSubcoreMesh(core_axis_name="c",
                                        subcore_axis_name="s",
                                        num_cores=_NC, num_subcores=_NS),
            scratch_types=[
                pltpu.VMEM((_NCH, _CH), jnp.int32),
                pltpu.VMEM((_NCH, _CH), jnp.int32),
                pltpu.VMEM((_NCH, _CH, ED), jnp.float32),
                pltpu.VMEM((_NCH, _CH, EM), jnp.float32),
                pltpu.SemaphoreType.DMA,
            ],
            compiler_params=pltpu.CompilerParams(use_tc_tiling_on_sc=False),
        ))
    return _gather_kernel_cache[0](dtab, mtab, didx, midx)


_R = 2048
_NCHUNK = B // _R


def _trunk_body(xin,
                Wc, bc, g96, be96,
                W11, b11, g11, be11, W12, b12, g12, be12, s1W, s1b,
                W21, b21, g21, be21, W22, b22, g22, be22, s2W, s2b,
                W31, b31, g31, be31, W32, b32, g32, be32, s3W, s3b,
                oW, ob, out, A, Bb, Rb):
    f32 = jnp.float32

    def mm(z, w):
        return lax.dot(z, w[...], preferred_element_type=f32)

    def finalize(s1, s2):
        mu = s1 * (1.0 / B)
        rstd = lax.rsqrt(s2 * (1.0 / B) - mu * mu + 1e-5)
        return mu, rstd

    def p1(i, c):
        s1, s2 = c
        y = mm(xin[pl.ds(i * _R, _R), :], Wc) + bc[...]
        A[pl.ds(i * _R, _R), :96] = y
        return (s1 + jnp.sum(y, 0, keepdims=True),
                s2 + jnp.sum(y * y, 0, keepdims=True))

    z1 = jnp.zeros((1, 96), f32)
    mu, rstd = finalize(*lax.fori_loop(0, _NCHUNK, p1, (z1, z1)))

    def mid_pass(src, Cs, mu, rstd, g, be, add_r, W, bv, dst, Cd,
                 sW=None, sb=None):
        def body(i, c):
            s1, s2 = c
            rows = pl.ds(i * _R, _R)
            z = g[...] * (src[rows, :Cs] - mu) * rstd + be[...]
            if add_r:
                z = z + Rb[rows, :Cs]
            z = jnp.maximum(z, 0.0)
            y = mm(z, W) + bv[...]
            dst[rows, :Cd] = y
            if sW is not None:
                Rb[rows, :Cd] = mm(z, sW) + sb[...]
            return (s1 + jnp.sum(y, 0, keepdims=True),
                    s2 + jnp.sum(y * y, 0, keepdims=True))
        zc = jnp.zeros((1, Cd), f32)
        return finalize(*lax.fori_loop(0, _NCHUNK, body, (zc, zc)))

    mu, rstd = mid_pass(A, 96, mu, rstd, g96, be96, False, W11, b11,
                        Bb, 128, s1W, s1b)
    mu, rstd = mid_pass(Bb, 128, mu, rstd, g11, be11, False, W12, b12,
                        A, 128)
    mu, rstd = mid_pass(A, 128, mu, rstd, g12, be12, True, W21, b21,
                        Bb, 64, s2W, s2b)
    mu, rstd = mid_pass(Bb, 64, mu, rstd, g21, be21, False, W22, b22,
                        A, 64)
    mu, rstd = mid_pass(A, 64, mu, rstd, g22, be22, True, W31, b31,
                        Bb, 32, s3W, s3b)
    mu, rstd = mid_pass(Bb, 32, mu, rstd, g31, be31, False, W32, b32,
                        A, 32)

    def pfin(i, _):
        rows = pl.ds(i * _R, _R)
        z = g32[...] * (A[rows, :32] - mu) * rstd + be32[...] + Rb[rows, :32]
        z = jnp.maximum(z, 0.0)
        out[rows, :] = mm(z, oW) + ob[...]
        return 0
    lax.fori_loop(0, _NCHUNK, pfin, 0)


def _trunk(xin, *ws):
    return pl.pallas_call(
        _trunk_body,
        out_shape=jax.ShapeDtypeStruct((B, 1), jnp.float32),
        scratch_shapes=[
            pltpu.VMEM((B, 128), jnp.float32),
            pltpu.VMEM((B, 128), jnp.float32),
            pltpu.VMEM((B, 128), jnp.float32),
        ],
    )(xin, *ws)


def kernel(diag, med, vitals, params):
    p = params
    d_rows, m_rows = _gather(
        p['diag_emb'], p['med_emb'],
        diag.reshape(_NW, _NCH, _CH), med.reshape(_NW, _NCH, _CH))
    xin = jnp.concatenate(
        [d_rows.reshape(B, ED), m_rows.reshape(B, EM), vitals], axis=1)

    dh, mh, vh = p['diag_head'], p['med_head'], p['vital_head']
    Wc = jnp.zeros((40, 96), jnp.float32)
    Wc = Wc.at[0:16, 0:32].set(dh['W'].T)
    Wc = Wc.at[16:32, 32:64].set(mh['W'].T)
    Wc = Wc.at[32:40, 64:96].set(vh['W'].T)
    bc = jnp.concatenate([dh['b'], mh['b'], vh['b']]).reshape(1, 96)
    g96 = jnp.concatenate([dh['g'], mh['g'], vh['g']]).reshape(1, 96)
    be96 = jnp.concatenate([dh['be'], mh['be'], vh['be']]).reshape(1, 96)

    def block_ws(blk):
        return (blk['W1'].T, blk['b1'].reshape(1, -1),
                blk['g1'].reshape(1, -1), blk['be1'].reshape(1, -1),
                blk['W2'].T, blk['b2'].reshape(1, -1),
                blk['g2'].reshape(1, -1), blk['be2'].reshape(1, -1),
                blk['skipW'].T, blk['skipb'].reshape(1, -1))

    ws = (Wc, bc, g96, be96,
          *block_ws(p['blocks'][0]), *block_ws(p['blocks'][1]),
          *block_ws(p['blocks'][2]),
          p['outW'].T, p['outb'].reshape(1, -1))
    return _trunk(xin, *ws)


# speedup vs baseline: 1.1400x; 1.1064x over previous
import jax
import jax.numpy as jnp
from jax import lax
from jax.experimental import pallas as pl
from jax.experimental.pallas import tpu as pltpu
from jax.experimental.pallas import tpu_sc as plsc

B = 16384
ED = 16
EM = 16

_NC, _NS = 2, 16
_NW = _NC * _NS
_BPW = B // _NW
_CH = 128
_NCH = _BPW // _CH


def _gather_body(dtab, mtab, didx, midx, out,
                 idx_d, idx_m, rows_d, rows_m, sem):
    wid = lax.axis_index("s") * _NC + lax.axis_index("c")
    base = wid * _BPW
    pltpu.sync_copy(didx.at[wid], idx_d)
    pltpu.sync_copy(midx.at[wid], idx_m)
    copies = []
    for j in range(_NCH):
        copies.append(pltpu.async_copy(
            dtab.at[idx_d.at[j]], rows_d.at[pl.ds(j * _CH, _CH), :], sem))
        copies.append(pltpu.async_copy(
            mtab.at[idx_m.at[j]], rows_m.at[pl.ds(j * _CH, _CH), :], sem))
    for c in copies:
        c.wait()
    pltpu.sync_copy(rows_d, out.at[pl.ds(base, _BPW), 0:ED])
    pltpu.sync_copy(rows_m, out.at[pl.ds(base, _BPW), ED:ED + EM])


_gather_kernel_cache = []


def _gather(dtab, mtab, didx, midx):
    if not _gather_kernel_cache:
        _gather_kernel_cache.append(pl.kernel(
            _gather_body,
            out_type=jax.ShapeDtypeStruct((B, ED + EM), jnp.float32),
            mesh=plsc.VectorSubcoreMesh(core_axis_name="c",
                                        subcore_axis_name="s",
                                        num_cores=_NC, num_subcores=_NS),
            scratch_types=[
                pltpu.VMEM((_NCH, _CH), jnp.int32),
                pltpu.VMEM((_NCH, _CH), jnp.int32),
                pltpu.VMEM((_BPW, ED), jnp.float32),
                pltpu.VMEM((_BPW, EM), jnp.float32),
                pltpu.SemaphoreType.DMA,
            ],
            compiler_params=pltpu.CompilerParams(use_tc_tiling_on_sc=False),
        ))
    return _gather_kernel_cache[0](dtab, mtab, didx, midx)


_R = 2048
_NCHUNK = B // _R


def _trunk_body(emb, vit,
                We, vW, bc, g96, be96,
                W11, b11, g11, be11, W12, b12, g12, be12, s1W, s1b,
                W21, b21, g21, be21, W22, b22, g22, be22, s2W, s2b,
                W31, b31, g31, be31, W32, b32, g32, be32, s3W, s3b,
                oW, ob, out, A, Bb, Rb):
    f32 = jnp.float32

    def mm(z, w):
        return lax.dot(z, w[...], preferred_element_type=f32)

    def finalize(s1, s2):
        mu = s1 * (1.0 / B)
        rstd = lax.rsqrt(s2 * (1.0 / B) - mu * mu + 1e-5)
        return mu, rstd

    def p1(i, c):
        s1, s2 = c
        rows = pl.ds(i * _R, _R)
        ye = mm(emb[rows, :], We)
        yv = mm(vit[rows, :], vW)
        y = jnp.concatenate([ye, yv], axis=1) + bc[...]
        A[rows, :96] = y
        return (s1 + jnp.sum(y, 0, keepdims=True),
                s2 + jnp.sum(y * y, 0, keepdims=True))

    z1 = jnp.zeros((1, 96), f32)
    mu, rstd = finalize(*lax.fori_loop(0, _NCHUNK, p1, (z1, z1)))

    def mid_pass(src, Cs, mu, rstd, g, be, add_r, W, bv, dst, Cd,
                 sW=None, sb=None):
        def body(i, c):
            s1, s2 = c
            rows = pl.ds(i * _R, _R)
            z = g[...] * (src[rows, :Cs] - mu) * rstd + be[...]
            if add_r:
                z = z + Rb[rows, :Cs]
            z = jnp.maximum(z, 0.0)
            y = mm(z, W) + bv[...]
            dst[rows, :Cd] = y
            if sW is not None:
                Rb[rows, :Cd] = mm(z, sW) + sb[...]
            return (s1 + jnp.sum(y, 0, keepdims=True),
                    s2 + jnp.sum(y * y, 0, keepdims=True))
        zc = jnp.zeros((1, Cd), f32)
        return finalize(*lax.fori_loop(0, _NCHUNK, body, (zc, zc)))

    mu, rstd = mid_pass(A, 96, mu, rstd, g96, be96, False, W11, b11,
                        Bb, 128, s1W, s1b)
    mu, rstd = mid_pass(Bb, 128, mu, rstd, g11, be11, False, W12, b12,
                        A, 128)
    mu, rstd = mid_pass(A, 128, mu, rstd, g12, be12, True, W21, b21,
                        Bb, 64, s2W, s2b)
    mu, rstd = mid_pass(Bb, 64, mu, rstd, g21, be21, False, W22, b22,
                        A, 64)
    mu, rstd = mid_pass(A, 64, mu, rstd, g22, be22, True, W31, b31,
                        Bb, 32, s3W, s3b)
    mu, rstd = mid_pass(Bb, 32, mu, rstd, g31, be31, False, W32, b32,
                        A, 32)

    def pfin(i, _):
        rows = pl.ds(i * _R, _R)
        z = g32[...] * (A[rows, :32] - mu) * rstd + be32[...] + Rb[rows, :32]
        z = jnp.maximum(z, 0.0)
        out[rows] = jnp.sum(z * oW[...], axis=1) + ob[0, 0]
        return 0
    lax.fori_loop(0, _NCHUNK, pfin, 0)


def _trunk(emb, vit, *ws):
    return pl.pallas_call(
        _trunk_body,
        out_shape=jax.ShapeDtypeStruct((B,), jnp.float32),
        scratch_shapes=[
            pltpu.VMEM((B, 128), jnp.float32),
            pltpu.VMEM((B, 128), jnp.float32),
            pltpu.VMEM((B, 128), jnp.float32),
        ],
    )(emb, vit, *ws)


def kernel(diag, med, vitals, params):
    p = params
    emb = _gather(
        p['diag_emb'], p['med_emb'],
        diag.reshape(_NW, _NCH, _CH), med.reshape(_NW, _NCH, _CH))

    dh, mh, vh = p['diag_head'], p['med_head'], p['vital_head']
    We = jnp.zeros((32, 64), jnp.float32)
    We = We.at[0:16, 0:32].set(dh['W'].T)
    We = We.at[16:32, 32:64].set(mh['W'].T)
    vWt = vh['W'].T
    bc = jnp.concatenate([dh['b'], mh['b'], vh['b']]).reshape(1, 96)
    g96 = jnp.concatenate([dh['g'], mh['g'], vh['g']]).reshape(1, 96)
    be96 = jnp.concatenate([dh['be'], mh['be'], vh['be']]).reshape(1, 96)

    def block_ws(blk):
        return (blk['W1'].T, blk['b1'].reshape(1, -1),
                blk['g1'].reshape(1, -1), blk['be1'].reshape(1, -1),
                blk['W2'].T, blk['b2'].reshape(1, -1),
                blk['g2'].reshape(1, -1), blk['be2'].reshape(1, -1),
                blk['skipW'].T, blk['skipb'].reshape(1, -1))

    ws = (We, vWt, bc, g96, be96,
          *block_ws(p['blocks'][0]), *block_ws(p['blocks'][1]),
          *block_ws(p['blocks'][2]),
          p['outW'].reshape(1, 32), p['outb'].reshape(1, 1))
    return _trunk(emb, vitals, *ws).reshape(B, 1)
